# baseline (speedup 1.0000x reference)
import functools

import jax
import jax.numpy as jnp
from jax import lax
from jax.experimental import pallas as pl
from jax.experimental.pallas import tpu as pltpu
from jax.experimental.pallas import tpu_sc as plsc

NUM_ROWS = 16384
NUM_COLS = 100
NC, NS, L = 1, 16, 16
NW = NC * NS
ROWS_PER_W = NUM_ROWS // NW
GROUPS = ROWS_PER_W // L
OUT_PER_W = ROWS_PER_W * 3

A0, A1, A2 = 3, 7, 12
C0, C1, C2 = 20, 45, 88

_OUT_IDX = jnp.array([[20], [45], [88]], dtype=jnp.int32)


def _tec_body(xt_hbm, cw_hbm, out_hbm,
              a0_v, a1_v, a2_v, c0_v, c1_v, c2_v,
              out_v, cw_v, sem):
    wid = lax.axis_index("s") * NC + lax.axis_index("c")
    base = wid * ROWS_PER_W
    rs = pl.ds(base, ROWS_PER_W)
    cps = [
        pltpu.async_copy(xt_hbm.at[pl.ds(A0, 1), rs], a0_v, sem),
        pltpu.async_copy(xt_hbm.at[pl.ds(A1, 1), rs], a1_v, sem),
        pltpu.async_copy(xt_hbm.at[pl.ds(A2, 1), rs], a2_v, sem),
        pltpu.async_copy(xt_hbm.at[pl.ds(C0, 1), rs], c0_v, sem),
        pltpu.async_copy(xt_hbm.at[pl.ds(C1, 1), rs], c1_v, sem),
        pltpu.async_copy(xt_hbm.at[pl.ds(C2, 1), rs], c2_v, sem),
        pltpu.async_copy(cw_hbm, cw_v, sem),
    ]
    for cp in cps:
        cp.wait()
    w = cw_v[...]

    def group(g, carry):
        s = pl.ds(g * L, L)
        a0 = a0_v[0, s]
        a1 = a1_v[0, s]
        a2 = a2_v[0, s]
        c0 = c0_v[0, s]
        c1 = -c1_v[0, s]
        c2 = c2_v[0, s]
        p = (1.0 + jnp.exp(a0)) * (1.0 + jnp.exp(-a1)) * (1.0 + jnp.exp(a2))
        m = jnp.maximum(c0, jnp.maximum(c1, c2))
        f0 = jnp.exp(c0 - m)
        f1 = jnp.exp(c1 - m)
        f2 = jnp.exp(c2 - m)
        d = w / ((f0 + f1 + f2) * p)
        oix = lax.iota(jnp.int32, L) * 3 + g * (L * 3)
        plsc.store_scatter(out_v, [oix], f0 * d)
        plsc.store_scatter(out_v, [oix + 1], -(f1 * d))
        plsc.store_scatter(out_v, [oix + 2], f2 * d)
        return carry

    lax.fori_loop(0, GROUPS, group, 0)

    pltpu.async_copy(
        out_v, out_hbm.at[pl.ds(wid * OUT_PER_W, OUT_PER_W)], sem
    ).wait()


@jax.jit
def _sc_boost(inputs_t, cw16):
    mesh = plsc.VectorSubcoreMesh(
        core_axis_name="c", subcore_axis_name="s", num_cores=NC
    )
    col = pltpu.VMEM((1, ROWS_PER_W), jnp.float32)
    f = functools.partial(
        pl.kernel,
        mesh=mesh,
        out_type=jax.ShapeDtypeStruct((NUM_ROWS * 3,), jnp.float32),
        compiler_params=pltpu.CompilerParams(
            needs_layout_passes=False,
            use_tc_tiling_on_sc=True,
            skip_device_barrier=True,
        ),
        scratch_types=[
            col, col, col, col, col, col,
            pltpu.VMEM((OUT_PER_W,), jnp.float32),
            pltpu.VMEM((L,), jnp.float32),
            pltpu.SemaphoreType.DMA,
        ],
    )(_tec_body)
    return f(inputs_t, cw16)


def kernel(inputs, clause_weight):
    cw16 = jnp.broadcast_to(jnp.reshape(clause_weight, ()), (L,))
    delta = _sc_boost(inputs.T, cw16)
    return (jnp.reshape(delta, (NUM_ROWS, 3)), _OUT_IDX)
